# Initial kernel scaffold; baseline (speedup 1.0000x reference)
import jax
import jax.numpy as jnp
from jax.experimental import pallas as pl


def kernel(syndrome, stab_id, cycle_id, stab_table, cycle_table, val_table):
    raise NotImplementedError("write your pallas kernel here")


# speedup vs baseline: 34.9164x; 34.9164x over previous
import jax
import jax.numpy as jnp
from jax.experimental import pallas as pl
from jax.experimental.pallas import tpu as pltpu


def _embed_body(stab_id_ref, cycle_id_ref, syn_ref, stab_ref, cyc_ref,
                val_ref, out_ref, base_ref):
    L, D = base_ref.shape

    @pl.when(pl.program_id(0) == 0)
    def _compute_base():
        ns = stab_ref.shape[0]
        nc = cyc_ref.shape[0]
        sid = stab_id_ref[...]
        cid = cycle_id_ref[...]
        oh_s = (sid == jax.lax.broadcasted_iota(jnp.int32, (L, ns), 1)
                ).astype(jnp.float32)
        oh_c = (cid == jax.lax.broadcasted_iota(jnp.int32, (L, nc), 1)
                ).astype(jnp.float32)
        base = jnp.dot(oh_s, stab_ref[...], preferred_element_type=jnp.float32)
        base += jnp.dot(oh_c, cyc_ref[...], preferred_element_type=jnp.float32)
        base_ref[...] = base + val_ref[0, :][None, :]

    syn = syn_ref[...].astype(jnp.float32)
    diff = val_ref[1, :] - val_ref[0, :]
    out_ref[...] = (base_ref[...][None, :, :]
                    + syn[:, :, None] * diff[None, None, :])


def kernel(syndrome, stab_id, cycle_id, stab_table, cycle_table, val_table):
    B, L = syndrome.shape
    D = stab_table.shape[1]
    BB = 32

    ns = stab_table.shape[0]
    ns_pad = ((ns + 127) // 128) * 128
    if ns_pad != ns:
        stab_table = jnp.pad(stab_table, ((0, ns_pad - ns), (0, 0)))

    sid = stab_id.astype(jnp.int32).reshape(L, 1)
    cid = cycle_id.astype(jnp.int32).reshape(L, 1)
    syn = syndrome.astype(jnp.int32)

    return pl.pallas_call(
        _embed_body,
        grid=(B // BB,),
        in_specs=[
            pl.BlockSpec((L, 1), lambda i: (0, 0)),
            pl.BlockSpec((L, 1), lambda i: (0, 0)),
            pl.BlockSpec((BB, L), lambda i: (i, 0)),
            pl.BlockSpec((ns_pad, D), lambda i: (0, 0)),
            pl.BlockSpec(cycle_table.shape, lambda i: (0, 0)),
            pl.BlockSpec((2, D), lambda i: (0, 0)),
        ],
        out_specs=pl.BlockSpec((BB, L, D), lambda i: (i, 0, 0)),
        out_shape=jax.ShapeDtypeStruct((B, L, D), jnp.float32),
        scratch_shapes=[pltpu.VMEM((L, D), jnp.float32)],
    )(sid, cid, syn, stab_table, cycle_table, val_table)
